# Initial kernel scaffold; baseline (speedup 1.0000x reference)
import jax
import jax.numpy as jnp
from jax.experimental import pallas as pl


def kernel(graph_emb, subgraph_emb, graph_ids, pos_ids, neg_ids):
    raise NotImplementedError("write your pallas kernel here")


# speedup vs baseline: 3.2071x; 3.2071x over previous
import functools

import jax
import jax.numpy as jnp
from jax import lax
from jax.experimental import pallas as pl
from jax.experimental.pallas import tpu as pltpu
from jax.experimental.pallas import tpu_sc as plsc

DIM = 64
B = 16384
NEG = 5

NC, NS, L = 2, 16, 16
NW = NC * NS
BW = B // NW
C = 128
NCHUNK = BW // C
GRP = C // L


def _softplus(x):
    u = jnp.exp(-jnp.abs(x))
    t = u / (u + 2.0)
    t2 = t * t
    p = t2 * (1.0 / 9.0) + (1.0 / 7.0)
    p = p * t2 + (1.0 / 5.0)
    p = p * t2 + (1.0 / 3.0)
    p = p * t2 + 1.0
    return jnp.maximum(x, 0.0) + 2.0 * t * p


def _body(gemb, semb, gids, pids, nids, out,
          gidx_v, pidx_v, nidx_v, g_buf, p_buf, n_buf, out_v, sem):
    wid = lax.axis_index("s") * NC + lax.axis_index("c")
    base = wid * BW

    def chunk_body(c, carry):
        ex0 = base + c * C
        pltpu.sync_copy(gids.at[pl.ds(ex0, C)], gidx_v)
        pltpu.sync_copy(pids.at[pl.ds(ex0, C)], pidx_v)
        pltpu.sync_copy(nids.at[pl.ds(ex0 * NEG, C * NEG)], nidx_v)

        def enq(g, _):
            gvec = gidx_v[pl.ds(g * L, L)]
            pvec = pidx_v[pl.ds(g * L, L)]
            for lane in range(L):
                e = g * L + lane
                pltpu.async_copy(gemb.at[pl.ds(gvec[lane], 1)],
                                 g_buf.at[pl.ds(e, 1)], sem)
                pltpu.async_copy(semb.at[pl.ds(pvec[lane], 1)],
                                 p_buf.at[pl.ds(e, 1)], sem)
            for sub in range(NEG):
                q0 = g * (L * NEG) + sub * L
                nvec = nidx_v[pl.ds(q0, L)]
                for lane in range(L):
                    q = q0 + lane
                    pltpu.async_copy(semb.at[pl.ds(nvec[lane], 1)],
                                     n_buf.at[pl.ds(q, 1)], sem)
            return 0

        lax.fori_loop(0, GRP, enq, 0)
        pltpu.make_async_copy(gemb.at[pl.ds(0, C)], g_buf, sem).wait()
        pltpu.make_async_copy(gemb.at[pl.ds(0, C)], p_buf, sem).wait()
        pltpu.make_async_copy(semb.at[pl.ds(0, C * NEG)], n_buf, sem).wait()

        def group_body(gi, gcarry):
            eidx = gi * L + lax.iota(jnp.int32, L)
            nidx0 = eidx * NEG
            zero = jnp.zeros((L,), jnp.float32)

            def d_body(dd, dc):
                ap, a0, a1, a2, a3, a4 = dc
                dsp = jnp.full((L,), dd, jnp.int32)
                gv = plsc.load_gather(g_buf, [eidx, dsp])
                pv = plsc.load_gather(p_buf, [eidx, dsp])
                n0 = plsc.load_gather(n_buf, [nidx0, dsp])
                n1 = plsc.load_gather(n_buf, [nidx0 + 1, dsp])
                n2 = plsc.load_gather(n_buf, [nidx0 + 2, dsp])
                n3 = plsc.load_gather(n_buf, [nidx0 + 3, dsp])
                n4 = plsc.load_gather(n_buf, [nidx0 + 4, dsp])
                return (ap + gv * pv, a0 + gv * n0, a1 + gv * n1,
                        a2 + gv * n2, a3 + gv * n3, a4 + gv * n4)

            ap, a0, a1, a2, a3, a4 = lax.fori_loop(
                0, DIM, d_body, (zero,) * 6)
            loss = (_softplus(-ap) + _softplus(a0) + _softplus(a1)
                    + _softplus(a2) + _softplus(a3) + _softplus(a4))
            plsc.store_scatter(out_v, [eidx], loss)
            return gcarry

        lax.fori_loop(0, GRP, group_body, 0)
        pltpu.sync_copy(out_v, out.at[pl.ds(ex0, C)])
        return carry

    lax.fori_loop(0, NCHUNK, chunk_body, 0)


_sc_call = functools.partial(
    pl.kernel,
    out_type=jax.ShapeDtypeStruct((B,), jnp.float32),
    mesh=plsc.VectorSubcoreMesh(core_axis_name="c", subcore_axis_name="s"),
    compiler_params=pltpu.CompilerParams(needs_layout_passes=False),
    scratch_types=[
        pltpu.VMEM((C,), jnp.int32),
        pltpu.VMEM((C,), jnp.int32),
        pltpu.VMEM((C * NEG,), jnp.int32),
        pltpu.VMEM((C, DIM), jnp.float32),
        pltpu.VMEM((C, DIM), jnp.float32),
        pltpu.VMEM((C * NEG, DIM), jnp.float32),
        pltpu.VMEM((C,), jnp.float32),
        pltpu.SemaphoreType.DMA,
    ],
)(_body)


def kernel(graph_emb, subgraph_emb, graph_ids, pos_ids, neg_ids):
    neg_flat = neg_ids.reshape(-1)
    return _sc_call(graph_emb, subgraph_emb, graph_ids, pos_ids, neg_flat)
